# Initial kernel scaffold; baseline (speedup 1.0000x reference)
import functools

import jax
import jax.numpy as jnp
from jax import lax
from jax.experimental import pallas as pl
from jax.experimental.pallas import tpu as pltpu
from jax.experimental.pallas import tpu_sc as plsc

N_DIM = 64
PADDING = 20
BATCH = 16384
LANES = 16


def _fasttext_body(x1_hbm, x2_hbm, wts_hbm, ez_hbm, ev_hbm, y_hbm,
                   idx1_v, idx2_v, wts_v, cols_v, u_v, v_v, y_v,
                   sem_w, sem_v, sem_z):
    nc = 2
    wid = lax.axis_index("s") * nc + lax.axis_index("c")
    bpw = BATCH // 32
    base = wid * bpw

    pltpu.sync_copy(x1_hbm.at[pl.ds(base, bpw)], idx1_v)
    pltpu.sync_copy(x2_hbm.at[pl.ds(base, bpw)], idx2_v)

    wts_cp = pltpu.async_copy(wts_hbm.at[idx1_v], wts_v, sem_w)
    v_cp = pltpu.async_copy(ev_hbm.at[idx2_v], v_v, sem_v)

    zeros16 = jnp.zeros((LANES,), jnp.float32)

    def zero_row(i, c):
        for k in range(N_DIM // LANES):
            u_v[i, pl.ds(k * LANES, LANES)] = zeros16
        return c

    lax.fori_loop(0, bpw, zero_row, 0)

    wts_cp.wait()

    iota = lax.iota(jnp.int32, LANES)

    def build_cols(i, c):
        rows = i * LANES + iota
        for p in range(PADDING):
            colp = jnp.full((LANES,), p, jnp.int32)
            vals = plsc.load_gather(wts_v, [rows, colp])
            cols_v[p, pl.ds(i * LANES, LANES)] = vals
        return c

    lax.fori_loop(0, bpw // LANES, build_cols, 0)

    bag_cps = []
    for p in range(PADDING):
        bag_cps.append(
            pltpu.async_copy(ez_hbm.at[cols_v.at[p]], u_v, sem_z, add=True))
    for cp in bag_cps:
        cp.wait()
    v_cp.wait()

    def dot_row(b, c):
        acc = zeros16
        for k in range(N_DIM // LANES):
            sl = pl.ds(k * LANES, LANES)
            acc = acc + u_v[b, sl] * v_v[b, sl]
        y_v[b] = jnp.sum(acc)
        return c

    lax.fori_loop(0, bpw, dot_row, 0)

    pltpu.sync_copy(y_v, y_hbm.at[pl.ds(base, bpw)])


@jax.jit
def kernel(x, word_to_sub, embedding_z, embedding_v):
    x1 = x[:, 0]
    x2 = x[:, 1]
    bpw = BATCH // 32
    mesh = plsc.VectorSubcoreMesh(core_axis_name="c", subcore_axis_name="s")
    kfn = pl.kernel(
        _fasttext_body,
        out_type=jax.ShapeDtypeStruct((BATCH,), jnp.float32),
        mesh=mesh,
        scratch_types=[
            pltpu.VMEM((bpw,), jnp.int32),
            pltpu.VMEM((bpw,), jnp.int32),
            pltpu.VMEM((bpw, PADDING), jnp.int32),
            pltpu.VMEM((PADDING, bpw), jnp.int32),
            pltpu.VMEM((bpw, N_DIM), jnp.float32),
            pltpu.VMEM((bpw, N_DIM), jnp.float32),
            pltpu.VMEM((bpw,), jnp.float32),
            pltpu.SemaphoreType.DMA,
            pltpu.SemaphoreType.DMA,
            pltpu.SemaphoreType.DMA,
        ],
    )
    return kfn(x1, x2, word_to_sub, embedding_z, embedding_v)


# speedup vs baseline: 6.0931x; 6.0931x over previous
import jax
import jax.numpy as jnp
from jax import lax
from jax.experimental import pallas as pl
from jax.experimental.pallas import tpu as pltpu
from jax.experimental.pallas import tpu_sc as plsc

N_SUBVOCAB = 50000
N_DIM = 64
PADDING = 20
PADW = 32
BATCH = 16384
LANES = 16
BPW = BATCH // 32
CHUNK = 128
NCH = BPW // CHUNK
MAX_OUT = 8


def _fasttext_body(x1_hbm, x2_hbm, wts_hbm, ez_hbm, ev_hbm, y_hbm,
                   idx1_v, idx2_v, wts_v, cols_v, u_v, v_v, y_v,
                   sem_w, sem_v, sem_z):
    nc = 2
    wid = lax.axis_index("s") * nc + lax.axis_index("c")
    base = wid * BPW

    for c in range(NCH):
        pltpu.sync_copy(x1_hbm.at[pl.ds(base + c * CHUNK, CHUNK)],
                        idx1_v.at[c])
        pltpu.sync_copy(x2_hbm.at[pl.ds(base + c * CHUNK, CHUNK)],
                        idx2_v.at[c])

    wts_cps = [
        pltpu.async_copy(wts_hbm.at[idx1_v.at[c]],
                         wts_v.at[pl.ds(c * CHUNK, CHUNK)], sem_w)
        for c in range(NCH)
    ]
    v_cps = [
        pltpu.async_copy(ev_hbm.at[idx2_v.at[c]],
                         v_v.at[pl.ds(c * CHUNK, CHUNK)], sem_v)
        for c in range(NCH)
    ]

    zeros16 = jnp.zeros((LANES,), jnp.float32)

    def zero_row(i, c):
        for k in range(N_DIM // LANES):
            u_v[i, pl.ds(k * LANES, LANES)] = zeros16
        return c

    lax.fori_loop(0, BPW, zero_row, 0)

    for cp in wts_cps:
        cp.wait()

    iota = lax.iota(jnp.int32, LANES)
    blocks_per_chunk = CHUNK // LANES

    def build_cols(i, c):
        rows = i * LANES + iota
        ch = i // blocks_per_chunk
        off = (i % blocks_per_chunk) * LANES
        for p in range(PADDING):
            colp = jnp.full((LANES,), p, jnp.int32)
            vals = plsc.load_gather(wts_v, [rows, colp])
            vals = jnp.minimum(jnp.maximum(vals, 0), N_SUBVOCAB)
            cols_v[p * NCH + ch, pl.ds(off, LANES)] = vals
        return c

    lax.fori_loop(0, BPW // LANES, build_cols, 0)

    pending = []
    for p in range(PADDING):
        for c in range(NCH):
            pending.append(
                pltpu.async_copy(ez_hbm.at[cols_v.at[p * NCH + c]],
                                 u_v.at[pl.ds(c * CHUNK, CHUNK)],
                                 sem_z, add=True))
            if len(pending) > MAX_OUT:
                pending.pop(0).wait()
    for cp in pending:
        cp.wait()
    for cp in v_cps:
        cp.wait()

    last_lane = iota == (LANES - 1)

    def dot_row(b, c):
        acc = zeros16
        for k in range(N_DIM // LANES):
            sl = pl.ds(k * LANES, LANES)
            acc = acc + u_v[b, sl] * v_v[b, sl]
        s = plsc.cumsum(acc)
        plsc.store_scatter(y_v, [jnp.full((LANES,), 0, jnp.int32) + b], s,
                           mask=last_lane)
        return c

    lax.fori_loop(0, BPW, dot_row, 0)

    pltpu.sync_copy(y_v, y_hbm.at[pl.ds(base, BPW)])


@jax.jit
def kernel(x, word_to_sub, embedding_z, embedding_v):
    x1 = x[:, 0]
    x2 = x[:, 1]
    wts32 = jnp.pad(word_to_sub, ((0, 0), (0, PADW - PADDING)))
    mesh = plsc.VectorSubcoreMesh(core_axis_name="c", subcore_axis_name="s",
                                  num_cores=2, num_subcores=16)
    kfn = pl.kernel(
        _fasttext_body,
        out_type=jax.ShapeDtypeStruct((BATCH,), jnp.float32),
        mesh=mesh,
        scratch_types=[
            pltpu.VMEM((NCH, CHUNK), jnp.int32),
            pltpu.VMEM((NCH, CHUNK), jnp.int32),
            pltpu.VMEM((BPW, PADW), jnp.int32),
            pltpu.VMEM((PADDING * NCH, CHUNK), jnp.int32),
            pltpu.VMEM((BPW, N_DIM), jnp.float32),
            pltpu.VMEM((BPW, N_DIM), jnp.float32),
            pltpu.VMEM((BPW,), jnp.float32),
            pltpu.SemaphoreType.DMA,
            pltpu.SemaphoreType.DMA,
            pltpu.SemaphoreType.DMA,
        ],
        compiler_params=pltpu.CompilerParams(needs_layout_passes=False,
                                             use_tc_tiling_on_sc=False),
    )
    return kfn(x1, x2, wts32, embedding_z, embedding_v)


# speedup vs baseline: 7.0803x; 1.1620x over previous
import jax
import jax.numpy as jnp
from jax import lax
from jax.experimental import pallas as pl
from jax.experimental.pallas import tpu as pltpu
from jax.experimental.pallas import tpu_sc as plsc

N_DIM = 64
PADDING = 20
PACK = 4
PACKW = PACK * PADDING
BATCH = 16384
LANES = 16
BPW = BATCH // 32
CHUNK = 128
NCH = BPW // CHUNK


def _fasttext_body(x_hbm, wts_hbm, ez_hbm, ev_hbm, y_hbm,
                   xs_v, idx1_v, idxb_v, idx2_v, wts_v, cols_v, u_v, v_v, y_v,
                   sem_x, sems_w, sems_v, sems_z):
    wid = lax.axis_index("s") * 2 + lax.axis_index("c")
    base = wid * BPW
    iota = lax.iota(jnp.int32, LANES)
    zeros16 = jnp.zeros((LANES,), jnp.float32)
    bpc = CHUNK // LANES

    pltpu.async_copy(x_hbm.at[pl.ds(base, BPW)], xs_v, sem_x).wait()

    def split_x(i, c):
        ch = i // bpc
        off = (i % bpc) * LANES
        rows = i * LANES + iota
        x1v = plsc.load_gather(xs_v, [rows, jnp.zeros((LANES,), jnp.int32)])
        x2v = plsc.load_gather(xs_v, [rows, jnp.ones((LANES,), jnp.int32)])
        idx1_v[ch, pl.ds(off, LANES)] = x1v
        idxb_v[ch, pl.ds(off, LANES)] = x1v >> 2
        idx2_v[ch, pl.ds(off, LANES)] = x2v
        return c

    lax.fori_loop(0, BPW // LANES, split_x, 0)

    wts_cps = [
        pltpu.async_copy(wts_hbm.at[idxb_v.at[c]],
                         wts_v.at[pl.ds(c * CHUNK, CHUNK)], sems_w.at[c])
        for c in range(NCH)
    ]
    v_cps = [
        pltpu.async_copy(ev_hbm.at[idx2_v.at[c]],
                         v_v.at[pl.ds(c * CHUNK, CHUNK)], sems_v.at[c])
        for c in range(NCH)
    ]

    def zero_row(i, c):
        for k in range(N_DIM // LANES):
            u_v[i, pl.ds(k * LANES, LANES)] = zeros16
        return c

    lax.fori_loop(0, BPW, zero_row, 0)

    bag_cps = [[] for _ in range(NCH)]
    for c in range(NCH):
        wts_cps[c].wait()

        def build_cols(j, carry):
            i = c * bpc + j
            rows = i * LANES + iota
            off = j * LANES
            x1v = idx1_v[c, pl.ds(off, LANES)]
            colbase = (x1v & (PACK - 1)) * PADDING
            for p in range(PADDING):
                vals = plsc.load_gather(wts_v, [rows, colbase + p])
                cols_v[p * NCH + c, pl.ds(off, LANES)] = vals
            return carry

        lax.fori_loop(0, bpc, build_cols, 0)
        for p in range(PADDING):
            bag_cps[c].append(
                pltpu.async_copy(ez_hbm.at[cols_v.at[p * NCH + c]],
                                 u_v.at[pl.ds(c * CHUNK, CHUNK)],
                                 sems_z.at[c], add=True))

    last_lane = iota == (LANES - 1)

    def dot_row(b, c):
        acc = zeros16
        for k in range(N_DIM // LANES):
            sl = pl.ds(k * LANES, LANES)
            acc = acc + u_v[b, sl] * v_v[b, sl]
        s = plsc.cumsum(acc)
        plsc.store_scatter(y_v, [jnp.full((LANES,), 0, jnp.int32) + b], s,
                           mask=last_lane)
        return c

    for c in range(NCH):
        for cp in bag_cps[c]:
            cp.wait()
        v_cps[c].wait()
        lax.fori_loop(c * CHUNK, (c + 1) * CHUNK, dot_row, 0)

    pltpu.sync_copy(y_v, y_hbm.at[pl.ds(base, BPW)])


@jax.jit
def kernel(x, word_to_sub, embedding_z, embedding_v):
    wts_packed = word_to_sub.reshape(word_to_sub.shape[0] // PACK, PACKW)
    mesh = plsc.VectorSubcoreMesh(core_axis_name="c", subcore_axis_name="s",
                                  num_cores=2, num_subcores=16)
    kfn = pl.kernel(
        _fasttext_body,
        out_type=jax.ShapeDtypeStruct((BATCH,), jnp.float32),
        mesh=mesh,
        scratch_types=[
            pltpu.VMEM((BPW, 2), jnp.int32),
            pltpu.VMEM((NCH, CHUNK), jnp.int32),
            pltpu.VMEM((NCH, CHUNK), jnp.int32),
            pltpu.VMEM((NCH, CHUNK), jnp.int32),
            pltpu.VMEM((BPW, PACKW), jnp.int32),
            pltpu.VMEM((PADDING * NCH, CHUNK), jnp.int32),
            pltpu.VMEM((BPW, N_DIM), jnp.float32),
            pltpu.VMEM((BPW, N_DIM), jnp.float32),
            pltpu.VMEM((BPW,), jnp.float32),
            pltpu.SemaphoreType.DMA,
            pltpu.SemaphoreType.DMA((NCH,)),
            pltpu.SemaphoreType.DMA((NCH,)),
            pltpu.SemaphoreType.DMA((NCH,)),
        ],
        compiler_params=pltpu.CompilerParams(needs_layout_passes=False,
                                             use_tc_tiling_on_sc=False),
    )
    return kfn(x, wts_packed, embedding_z, embedding_v)


# speedup vs baseline: 7.1272x; 1.0066x over previous
import jax
import jax.numpy as jnp
from jax import lax
from jax.experimental import pallas as pl
from jax.experimental.pallas import tpu as pltpu
from jax.experimental.pallas import tpu_sc as plsc

N_DIM = 64
PADDING = 20
PACK = 4
PACKW = PACK * PADDING
BATCH = 16384
LANES = 16
BPW = BATCH // 32
CHUNK = 128
NCH = BPW // CHUNK


def _fasttext_body(x_hbm, wts_hbm, ez_hbm, ev_hbm, y_hbm,
                   xs_v, idx1_v, idxb_v, idx2_v, wts_v, cols_v, u_v, v_v, y_v,
                   sem_x, sems_w, sems_v, sems_z):
    wid = lax.axis_index("s") * 2 + lax.axis_index("c")
    base = wid * BPW
    iota = lax.iota(jnp.int32, LANES)
    zeros16 = jnp.zeros((LANES,), jnp.float32)
    bpc = CHUNK // LANES

    pltpu.async_copy(x_hbm.at[pl.ds(base, BPW)], xs_v, sem_x).wait()

    def split_x(i, c):
        ch = i // bpc
        off = (i % bpc) * LANES
        rows = i * LANES + iota
        x1v = plsc.load_gather(xs_v, [rows, jnp.zeros((LANES,), jnp.int32)])
        x2v = plsc.load_gather(xs_v, [rows, jnp.ones((LANES,), jnp.int32)])
        idx1_v[ch, pl.ds(off, LANES)] = x1v
        idxb_v[ch, pl.ds(off, LANES)] = x1v >> 2
        idx2_v[ch, pl.ds(off, LANES)] = x2v
        return c

    lax.fori_loop(0, BPW // LANES, split_x, 0)

    def fire_rows(c, _):
        pltpu.async_copy(wts_hbm.at[idxb_v.at[c]],
                         wts_v.at[pl.ds(c * CHUNK, CHUNK)], sems_w.at[c])
        pltpu.async_copy(ev_hbm.at[idx2_v.at[c]],
                         v_v.at[pl.ds(c * CHUNK, CHUNK)], sems_v.at[c])
        return _

    lax.fori_loop(0, NCH, fire_rows, 0)

    def zero_row(i, c):
        for k in range(N_DIM // LANES):
            u_v[i, pl.ds(k * LANES, LANES)] = zeros16
        return c

    lax.fori_loop(0, BPW, zero_row, 0)

    def stage_chunk(c, _):
        pltpu.make_async_copy(wts_hbm.at[idxb_v.at[c]],
                              wts_v.at[pl.ds(c * CHUNK, CHUNK)],
                              sems_w.at[c]).wait()

        def build_cols(j, carry):
            i = c * bpc + j
            rows = i * LANES + iota
            off = j * LANES
            x1v = idx1_v[c, pl.ds(off, LANES)]
            colbase = (x1v & (PACK - 1)) * PADDING

            def one_col(p, cc):
                vals = plsc.load_gather(wts_v, [rows, colbase + p])
                cols_v[p * NCH + c, pl.ds(off, LANES)] = vals
                return cc

            lax.fori_loop(0, PADDING, one_col, 0)
            return carry

        lax.fori_loop(0, bpc, build_cols, 0)

        def fire_bag(p, cc):
            pltpu.async_copy(ez_hbm.at[cols_v.at[p * NCH + c]],
                             u_v.at[pl.ds(c * CHUNK, CHUNK)],
                             sems_z.at[c], add=True)
            return cc

        lax.fori_loop(0, PADDING, fire_bag, 0)
        return _

    lax.fori_loop(0, NCH, stage_chunk, 0)

    last_lane = iota == (LANES - 1)

    def dot_row(b, c):
        acc = zeros16
        for k in range(N_DIM // LANES):
            sl = pl.ds(k * LANES, LANES)
            acc = acc + u_v[b, sl] * v_v[b, sl]
        s = plsc.cumsum(acc)
        plsc.store_scatter(y_v, [jnp.full((LANES,), 0, jnp.int32) + b], s,
                           mask=last_lane)
        return c

    def finish_chunk(c, _):
        def drain(p, cc):
            pltpu.make_async_copy(ez_hbm.at[cols_v.at[p * NCH + c]],
                                  u_v.at[pl.ds(c * CHUNK, CHUNK)],
                                  sems_z.at[c]).wait()
            return cc

        lax.fori_loop(0, PADDING, drain, 0)
        pltpu.make_async_copy(ev_hbm.at[idx2_v.at[c]],
                              v_v.at[pl.ds(c * CHUNK, CHUNK)],
                              sems_v.at[c]).wait()
        lax.fori_loop(c * CHUNK, (c + 1) * CHUNK, dot_row, 0)
        return _

    lax.fori_loop(0, NCH, finish_chunk, 0)

    pltpu.sync_copy(y_v, y_hbm.at[pl.ds(base, BPW)])


@jax.jit
def kernel(x, word_to_sub, embedding_z, embedding_v):
    wts_packed = word_to_sub.reshape(word_to_sub.shape[0] // PACK, PACKW)
    mesh = plsc.VectorSubcoreMesh(core_axis_name="c", subcore_axis_name="s",
                                  num_cores=2, num_subcores=16)
    kfn = pl.kernel(
        _fasttext_body,
        out_type=jax.ShapeDtypeStruct((BATCH,), jnp.float32),
        mesh=mesh,
        scratch_types=[
            pltpu.VMEM((BPW, 2), jnp.int32),
            pltpu.VMEM((NCH, CHUNK), jnp.int32),
            pltpu.VMEM((NCH, CHUNK), jnp.int32),
            pltpu.VMEM((NCH, CHUNK), jnp.int32),
            pltpu.VMEM((BPW, PACKW), jnp.int32),
            pltpu.VMEM((PADDING * NCH, CHUNK), jnp.int32),
            pltpu.VMEM((BPW, N_DIM), jnp.float32),
            pltpu.VMEM((BPW, N_DIM), jnp.float32),
            pltpu.VMEM((BPW,), jnp.float32),
            pltpu.SemaphoreType.DMA,
            pltpu.SemaphoreType.DMA((NCH,)),
            pltpu.SemaphoreType.DMA((NCH,)),
            pltpu.SemaphoreType.DMA((NCH,)),
        ],
        compiler_params=pltpu.CompilerParams(needs_layout_passes=False,
                                             use_tc_tiling_on_sc=False),
    )
    return kfn(x, wts_packed, embedding_z, embedding_v)
